# baseline (speedup 1.0000x reference)
import functools

import jax
import jax.numpy as jnp
from jax import lax
from jax.experimental import pallas as pl
from jax.experimental.pallas import tpu as pltpu
from jax.experimental.pallas import tpu_sc as plsc

N_NODE = 50000
E = 800000
D_IN = 128
H = 64

RB = 512
NR = 98
NPAD = NR * RB
DUMMY = RB

UNITS = E // 16
U_PER_W = UNITS // 32
CH_E = 2048
EPAD_IN = E + CH_E
SD = 128
SROW = SD + 16
EP2 = E + NR * 32 * 16 + CH_E

S2_CH = 4096
G = 128

_SC_PARAMS = pltpu.CompilerParams(
    needs_layout_passes=False, use_tc_tiling_on_sc=False)


def _mesh():
    return plsc.VectorSubcoreMesh(core_axis_name="c", subcore_axis_name="s")


def _splat(v, dtype=jnp.int32):
    return jnp.zeros((16,), dtype) + v


def _make_hist():
    @functools.partial(
        pl.kernel, mesh=_mesh(), compiler_params=_SC_PARAMS,
        out_type=[jax.ShapeDtypeStruct((256, 16), jnp.float32)],
        scratch_types=[pltpu.VMEM((CH_E,), jnp.int32),
                       pltpu.VMEM((8, 16), jnp.float32),
                       pltpu.SemaphoreType.DMA])
    def hist(dst_hbm, hist_hbm, dbuf, hcnt, sem):
        wid = lax.axis_index("s") * 2 + lax.axis_index("c")
        off_u = wid * U_PER_W + jnp.minimum(wid, 16)
        nu = U_PER_W + jnp.where(wid < 16, 1, 0)
        lane = lax.iota(jnp.int32, 16)
        zf = jnp.zeros((16,), jnp.float32)
        for i in range(8):
            hcnt[i, :] = zf

        nch = (nu + 127) // 128

        def chunk(j, _):
            ubase = off_u + j * 128
            pltpu.sync_copy(dst_hbm.at[pl.ds(ubase * 16, CH_E)], dbuf)
            uvalid = jnp.minimum(128, nu - j * 128)

            def unit(u, _):
                d = dbuf[pl.ds(u * 16, 16)]
                rngv = d >> 9
                for e in range(16):
                    rg = rngv[e]
                    cr = rg >> 4
                    oh = (lane == (rg & 15)).astype(jnp.float32)
                    plsc.addupdate(hcnt.at[cr, :], oh)
                return 0
            lax.fori_loop(0, uvalid, unit, 0)
            return 0

        lax.fori_loop(0, nch, chunk, 0)
        pltpu.sync_copy(hcnt.at[pl.ds(0, 8)], hist_hbm.at[pl.ds(wid * 8, 8)])

    return hist


def _make_place():
    @functools.partial(
        pl.kernel, mesh=_mesh(), compiler_params=_SC_PARAMS,
        out_type=[jax.ShapeDtypeStruct((EP2,), jnp.int32),
                  jax.ShapeDtypeStruct((EP2,), jnp.int32)],
        scratch_types=[pltpu.VMEM((CH_E,), jnp.int32),
                       pltpu.VMEM((CH_E,), jnp.int32),
                       pltpu.VMEM((NR * SROW + 16,), jnp.int32),
                       pltpu.VMEM((NR * SROW + 16,), jnp.int32),
                       pltpu.VMEM((1600,), jnp.int32),
                       pltpu.VMEM((1600,), jnp.int32),
                       pltpu.SemaphoreType.DMA,
                       pltpu.SemaphoreType.DMA,
                       pltpu.SemaphoreType.DMA])
    def place(src_hbm, dst_hbm, starts_hbm, psrc_hbm, pdl_hbm,
              sbuf, dbuf, stg_s, stg_d, fill, wcur, sem_i, sem_fs, sem_fd):
        wid = lax.axis_index("s") * 2 + lax.axis_index("c")
        off_u = wid * U_PER_W + jnp.minimum(wid, 16)
        nu = U_PER_W + jnp.where(wid < 16, 1, 0)
        lane = lax.iota(jnp.int32, 16)
        lane0 = lane == 0
        zi = jnp.zeros((16,), jnp.int32)

        def zfill(i, _):
            fill[pl.ds(i * 16, 16)] = zi
            return 0
        lax.fori_loop(0, 100, zfill, 0)
        pltpu.sync_copy(starts_hbm.at[pl.ds(wid * 1600, 1600)], wcur)

        def flush(rg, nun):
            co = pl.multiple_of(wcur[pl.ds(rg * 16, 16)][0], 16)

            def fire(u2, _):
                pltpu.async_copy(
                    stg_s.at[pl.ds(rg * SROW + u2 * 16, 16)],
                    psrc_hbm.at[pl.ds(co + u2 * 16, 16)], sem_fs)
                pltpu.async_copy(
                    stg_d.at[pl.ds(rg * SROW + u2 * 16, 16)],
                    pdl_hbm.at[pl.ds(co + u2 * 16, 16)], sem_fd)
                return 0
            lax.fori_loop(0, nun, fire, 0)

            def drain(u2, _):
                pltpu.make_async_copy(
                    stg_s.at[pl.ds(rg * SROW + u2 * 16, 16)],
                    psrc_hbm.at[pl.ds(co + u2 * 16, 16)], sem_fs).wait()
                pltpu.make_async_copy(
                    stg_d.at[pl.ds(rg * SROW + u2 * 16, 16)],
                    pdl_hbm.at[pl.ds(co + u2 * 16, 16)], sem_fd).wait()
                return 0
            lax.fori_loop(0, nun, drain, 0)
            plsc.store_scatter(wcur, [_splat(rg * 16)],
                               _splat(co + nun * 16), mask=lane0)

        nch = (nu + 127) // 128

        def chunk(j, _):
            ubase = off_u + j * 128
            pltpu.sync_copy(src_hbm.at[pl.ds(ubase * 16, CH_E)], sbuf)
            pltpu.sync_copy(dst_hbm.at[pl.ds(ubase * 16, CH_E)], dbuf)
            uvalid = jnp.minimum(128, nu - j * 128)

            def unit(u, _):
                u16 = u * 16
                d = dbuf[pl.ds(u16, 16)]
                rgv = d >> 9
                srg, perm = plsc.sort_key_val(rgv, lane)
                prev = srg.at[jnp.maximum(lane - 1, 0)].get(
                    mode="promise_in_bounds")
                runstart = (lane == 0) | (srg != prev)
                sidx = jnp.where(runstart, lane, 0)
                runpos = lane - plsc.cummax(sidx)
                base = plsc.load_gather(fill, [srg * 16])
                posv = srg * SROW + base + runpos
                sv = plsc.load_gather(sbuf, [u16 + perm])
                dlv = plsc.load_gather(dbuf, [u16 + perm]) & (RB - 1)
                plsc.store_scatter(stg_s, [posv], sv)
                plsc.store_scatter(stg_d, [posv], dlv)
                nxt = srg.at[jnp.minimum(lane + 1, 15)].get(
                    mode="promise_in_bounds")
                lastrun = jnp.where((lane == 15) | (srg != nxt), 1, 0)
                newf = base + runpos + 1
                plsc.store_scatter(fill, [srg * 16], newf,
                                   mask=lastrun == 1)
                mx = lax.reduce_max(newf, axes=(0,))

                @pl.when(mx >= SD)
                def _():
                    for e in range(16):
                        rg_e = srg[e]

                        @pl.when((lastrun[e] == 1) & (newf[e] >= SD))
                        def _():
                            flush(rg_e, SD // 16)
                            stg_s[pl.ds(rg_e * SROW, 16)] = \
                                stg_s[pl.ds(rg_e * SROW + SD, 16)]
                            stg_d[pl.ds(rg_e * SROW, 16)] = \
                                stg_d[pl.ds(rg_e * SROW + SD, 16)]
                            plsc.store_scatter(fill, [_splat(rg_e * 16)],
                                               _splat(newf[e] - SD),
                                               mask=lane0)
                return 0
            lax.fori_loop(0, uvalid, unit, 0)
            return 0

        lax.fori_loop(0, nch, chunk, 0)

        def tail(rg, _):
            f = fill[pl.ds(rg * 16, 16)][0]
            stg_s[pl.ds(rg * SROW + f, 16)] = zi
            stg_d[pl.ds(rg * SROW + f, 16)] = _splat(DUMMY)

            @pl.when(f > 0)
            def _():
                flush(rg, (f + 15) >> 4)
            return 0
        lax.fori_loop(0, NR, tail, 0)

    return place


def _make_segsum(with_counts):
    @functools.partial(
        pl.kernel, mesh=_mesh(), compiler_params=_SC_PARAMS,
        out_type=[jax.ShapeDtypeStruct((NPAD, H), jnp.float32),
                  jax.ShapeDtypeStruct((NR * 32, 16), jnp.float32)],
        scratch_types=[pltpu.VMEM((S2_CH + 128,), jnp.int32),
                       pltpu.VMEM((S2_CH + 128,), jnp.int32),
                       pltpu.VMEM((2, G, H), jnp.float32),
                       pltpu.VMEM((RB + 1, H), jnp.float32),
                       pltpu.VMEM((40, 16), jnp.float32),
                       pltpu.VMEM((1600,), jnp.int32),
                       pltpu.SemaphoreType.DMA,
                       pltpu.SemaphoreType.DMA,
                       pltpu.SemaphoreType.DMA])
    def seg(x_hbm, psrc_hbm, pdl_hbm, bounds_hbm, sums_hbm, cnts_hbm,
            sbuf, dbuf, rows, acc, cnt16, bbuf, sem_i, sg0, sg1):
        wid = lax.axis_index("s") * 2 + lax.axis_index("c")
        lane = lax.iota(jnp.int32, 16)
        zf = jnp.zeros((16,), jnp.float32)
        zi = jnp.zeros((16,), jnp.int32)
        pltpu.sync_copy(bounds_hbm, bbuf)

        def issue(g, b):
            pltpu.async_copy(
                x_hbm.at[sbuf.at[pl.ds(g * G, G)]], rows.at[b],
                sg0 if b == 0 else sg1)

        def wait(b):
            pltpu.make_async_copy(
                x_hbm.at[pl.ds(0, G)], rows.at[b],
                sg0 if b == 0 else sg1).wait()

        def accum(g, b):
            for sub in range(G // 16):
                dvv = dbuf[pl.ds(g * G + sub * 16, 16)]
                for e in range(16):
                    dloc = dvv[e]
                    rr = sub * 16 + e
                    for f in range(H // 16):
                        sl = pl.ds(f * 16, 16)
                        plsc.addupdate(acc.at[dloc, sl], rows[b, rr, sl])
                    if with_counts:
                        oh = (lane == (dloc & 15)).astype(jnp.float32)
                        plsc.addupdate(cnt16.at[dloc >> 4, :], oh)

        def process_range(r):
            b0 = pl.multiple_of(bbuf[pl.ds(r * 16, 16)][0], 16)
            b1 = pl.multiple_of(bbuf[pl.ds(r * 16 + 16, 16)][0], 16)

            def zero_body(i, _):
                for f in range(H // 16):
                    acc[i, pl.ds(f * 16, 16)] = zf
                return 0
            lax.fori_loop(0, RB + 1, zero_body, 0)

            def zero_cnt(i, _):
                cnt16[i, :] = zf
                return 0
            lax.fori_loop(0, 40, zero_cnt, 0)

            nch = (b1 - b0 + S2_CH - 1) // S2_CH

            def chunk(j, _):
                base = b0 + j * S2_CH
                pltpu.sync_copy(psrc_hbm.at[pl.ds(base, S2_CH)],
                                sbuf.at[pl.ds(0, S2_CH)])
                pltpu.sync_copy(pdl_hbm.at[pl.ds(base, S2_CH)],
                                dbuf.at[pl.ds(0, S2_CH)])
                nvalid = jnp.minimum(S2_CH, b1 - base)
                for q in range(G // 16 - 1):
                    sbuf[pl.ds(nvalid + q * 16, 16)] = zi
                    dbuf[pl.ds(nvalid + q * 16, 16)] = _splat(DUMMY)
                ngrp = (nvalid + G - 1) // G

                issue(0, 0)

                def gpair(gp, _):
                    g0 = 2 * gp

                    @pl.when(g0 + 1 < ngrp)
                    def _():
                        issue(g0 + 1, 1)
                    wait(0)
                    accum(g0, 0)

                    @pl.when(g0 + 1 < ngrp)
                    def _():
                        @pl.when(g0 + 2 < ngrp)
                        def _():
                            issue(g0 + 2, 0)
                        wait(1)
                        accum(g0 + 1, 1)
                    return 0
                lax.fori_loop(0, (ngrp + 1) // 2, gpair, 0)
                return 0
            lax.fori_loop(0, nch, chunk, 0)

            pltpu.sync_copy(acc.at[pl.ds(0, RB)],
                            sums_hbm.at[pl.ds(r * RB, RB)])
            pltpu.sync_copy(cnt16.at[pl.ds(0, 32)],
                            cnts_hbm.at[pl.ds(r * 32, 32)])

        def kloop(k, _):
            r = wid + 32 * k

            @pl.when(r < NR)
            def _():
                process_range(r)
            return 0
        lax.fori_loop(0, 4, kloop, 0)

    return seg


_hist = _make_hist()
_place = _make_place()
_segsum_wc = _make_segsum(True)
_segsum_nc = _make_segsum(False)


def _partition(src, dst):
    src_p = jnp.pad(src, (0, EPAD_IN - E))
    dst_p = jnp.pad(dst, (0, EPAD_IN - E))
    hist = _hist(dst_p)[0]
    cnt_wr = hist.reshape(32, 8, 16)[:, :7, :].reshape(32, 112)[:, :NR]
    cnt_wr = cnt_wr.astype(jnp.int32)
    pc = ((cnt_wr + 15) // 16) * 16
    p_r = pc.sum(axis=0)
    range_starts = jnp.concatenate(
        [jnp.zeros((1,), jnp.int32), jnp.cumsum(p_r)]).astype(jnp.int32)
    starts_wr = range_starts[None, :NR] + (jnp.cumsum(pc, axis=0) - pc)
    starts = jnp.zeros((32, 100, 16), jnp.int32).at[:, :NR, 0].set(starts_wr)
    bounds = jnp.full((100, 16), range_starts[NR], jnp.int32)
    bounds = bounds.at[:NR + 1, 0].set(range_starts)
    psrc, pdl = _place(src_p, dst_p, starts.reshape(-1))
    bounds = bounds.reshape(-1)
    return psrc, pdl, bounds


def _seg(x, part):
    psrc, pdl, bounds = part
    sums, cnts = _segsum_wc(x, psrc, pdl, bounds)
    counts = cnts.reshape(NPAD)[:N_NODE].reshape(N_NODE, 1)
    return sums[:N_NODE], counts


def _seg_nc(x, part):
    psrc, pdl, bounds = part
    return _segsum_nc(x, psrc, pdl, bounds)[0][:N_NODE]


BN = 2000
NB = N_NODE // BN


def _enc_body(x_ref, w_ref, b_ref, o_ref):
    o_ref[...] = jnp.maximum(
        jnp.dot(x_ref[...], w_ref[...], preferred_element_type=jnp.float32)
        + b_ref[...], 0.0)


def _encode(x, w, b):
    return pl.pallas_call(
        _enc_body,
        grid=(NB,),
        in_specs=[
            pl.BlockSpec((BN, D_IN), lambda i: (i, 0)),
            pl.BlockSpec((D_IN, H), lambda i: (0, 0)),
            pl.BlockSpec((1, H), lambda i: (0, 0)),
        ],
        out_specs=pl.BlockSpec((BN, H), lambda i: (i, 0)),
        out_shape=jax.ShapeDtypeStruct((N_NODE, H), jnp.float32),
    )(x, w, b.reshape(1, H))


def _inv_counts(c_ref):
    return 1.0 / jnp.maximum(c_ref[...], 1.0)


def _l1bus_body(sbb_ref, ibb_ref, sdb_ref, idb_ref, xb_ref,
                wlbb_ref, wrbb_ref, wldb_ref, wrdb_ref, bias_ref, o_ref):
    mbb = (sbb_ref[...] * _inv_counts(ibb_ref)) @ wlbb_ref[...]
    mdb = (sdb_ref[...] * _inv_counts(idb_ref)) @ wldb_ref[...]
    xr = xb_ref[...] @ (wrbb_ref[...] + wrdb_ref[...])
    o_ref[...] = jnp.maximum(mbb + mdb + xr + bias_ref[...], 0.0)


def _l1dev_body(s_ref, i_ref, x_ref, wl_ref, wr_ref, bias_ref, o_ref):
    m = (s_ref[...] * _inv_counts(i_ref)) @ wl_ref[...]
    o_ref[...] = jnp.maximum(m + x_ref[...] @ wr_ref[...] + bias_ref[...], 0.0)


def _l2head_body(sbb_ref, ibb_ref, sdb_ref, idb_ref, xb_ref,
                 wlbb_ref, wrbb_ref, wldb_ref, wrdb_ref, bias_ref,
                 hw_ref, hb_ref, o_ref, acc_ref):
    i = pl.program_id(0)

    @pl.when(i == 0)
    def _init():
        acc_ref[...] = jnp.zeros_like(acc_ref)

    mbb = (sbb_ref[...] * _inv_counts(ibb_ref)) @ wlbb_ref[...]
    mdb = (sdb_ref[...] * _inv_counts(idb_ref)) @ wldb_ref[...]
    xr = xb_ref[...] @ (wrbb_ref[...] + wrdb_ref[...])
    blk = jnp.maximum(mbb + mdb + xr + bias_ref[...], 0.0)
    acc_ref[...] += jnp.sum(blk, axis=0, keepdims=True)

    @pl.when(i == NB - 1)
    def _fin():
        o_ref[...] = (acc_ref[...] / N_NODE) @ hw_ref[...] + hb_ref[...]


_SPEC_NH = pl.BlockSpec((BN, H), lambda i: (i, 0))
_SPEC_C = pl.BlockSpec((BN, 1), lambda i: (i, 0))
_SPEC_W = pl.BlockSpec((H, H), lambda i: (0, 0))
_SPEC_B = pl.BlockSpec((1, H), lambda i: (0, 0))


def _layer_bus(sbb, ibb, sdb, idb, xb, wlbb, wrbb, wldb, wrdb, bias):
    return pl.pallas_call(
        _l1bus_body,
        grid=(NB,),
        in_specs=[_SPEC_NH, _SPEC_C, _SPEC_NH, _SPEC_C, _SPEC_NH,
                  _SPEC_W, _SPEC_W, _SPEC_W, _SPEC_W, _SPEC_B],
        out_specs=_SPEC_NH,
        out_shape=jax.ShapeDtypeStruct((N_NODE, H), jnp.float32),
    )(sbb, ibb, sdb, idb, xb, wlbb, wrbb, wldb, wrdb, bias.reshape(1, H))


def _layer_dev(s, iv, x, wl, wr, bias):
    return pl.pallas_call(
        _l1dev_body,
        grid=(NB,),
        in_specs=[_SPEC_NH, _SPEC_C, _SPEC_NH, _SPEC_W, _SPEC_W, _SPEC_B],
        out_specs=_SPEC_NH,
        out_shape=jax.ShapeDtypeStruct((N_NODE, H), jnp.float32),
    )(s, iv, x, wl, wr, bias.reshape(1, H))


def _layer2_head(sbb, ibb, sdb, idb, xb2, wlbb, wrbb, wldb, wrdb, bias,
                 hw, hb):
    return pl.pallas_call(
        _l2head_body,
        grid=(NB,),
        in_specs=[_SPEC_NH, _SPEC_C, _SPEC_NH, _SPEC_C, _SPEC_NH,
                  _SPEC_W, _SPEC_W, _SPEC_W, _SPEC_W, _SPEC_B,
                  pl.BlockSpec((H, 1), lambda i: (0, 0)),
                  pl.BlockSpec((1, 1), lambda i: (0, 0))],
        out_specs=pl.BlockSpec((1, 1), lambda i: (0, 0)),
        out_shape=jax.ShapeDtypeStruct((1, 1), jnp.float32),
        scratch_shapes=[pltpu.VMEM((1, H), jnp.float32)],
    )(sbb, ibb, sdb, idb, xb2, wlbb, wrbb, wldb, wrdb, bias.reshape(1, H),
      hw, hb.reshape(1, 1))


def kernel(x_bus, x_device, edge_index_bb, edge_index_bd, edge_index_db,
           bus_W, bus_b, dev_W, dev_b,
           Wl_bb1, bl_bb1, Wr_bb1, Wl_bb2, bl_bb2, Wr_bb2,
           Wl_bd1, bl_bd1, Wr_bd1, Wl_bd2, bl_bd2, Wr_bd2,
           Wl_db1, bl_db1, Wr_db1, Wl_db2, bl_db2, Wr_db2,
           head_W, head_b):
    xb = _encode(x_bus, bus_W, bus_b)
    xd = _encode(x_device, dev_W, dev_b)

    part_bb = _partition(edge_index_bb[0], edge_index_bb[1])
    part_bd = _partition(edge_index_bd[0], edge_index_bd[1])
    part_db = _partition(edge_index_db[0], edge_index_db[1])

    s_bb1, c_bb = _seg(xb, part_bb)
    s_bd1, c_bd = _seg(xb, part_bd)
    s_db1, c_db = _seg(xd, part_db)

    xb2 = _layer_bus(s_bb1, c_bb, s_db1, c_db, xb,
                     Wl_bb1, Wr_bb1, Wl_db1, Wr_db1, bl_bb1 + bl_db1)
    xd2 = _layer_dev(s_bd1, c_bd, xd, Wl_bd1, Wr_bd1, bl_bd1)

    s_bb2 = _seg_nc(xb2, part_bb)
    s_db2 = _seg_nc(xd2, part_db)

    return _layer2_head(s_bb2, c_bb, s_db2, c_db, xb2,
                        Wl_bb2, Wr_bb2, Wl_db2, Wr_db2, bl_bb2 + bl_db2,
                        head_W, head_b)


# speedup vs baseline: 1.0314x; 1.0314x over previous
import functools

import jax
import jax.numpy as jnp
from jax import lax
from jax.experimental import pallas as pl
from jax.experimental.pallas import tpu as pltpu
from jax.experimental.pallas import tpu_sc as plsc

N_NODE = 50000
E = 800000
D_IN = 128
H = 64

RB = 512
NR = 98
NPAD = NR * RB
DUMMY = RB

UNITS = E // 16
U_PER_W = UNITS // 32
CH_E = 2048
EPAD_IN = E + CH_E
SD = 128
SROW = SD + 16
EP2 = E + NR * 32 * 16 + CH_E

S2_CH = 2048
G = 64

_SC_PARAMS = pltpu.CompilerParams(
    needs_layout_passes=False, use_tc_tiling_on_sc=False)


def _mesh():
    return plsc.VectorSubcoreMesh(core_axis_name="c", subcore_axis_name="s")


def _splat(v, dtype=jnp.int32):
    return jnp.zeros((16,), dtype) + v


def _make_hist():
    @functools.partial(
        pl.kernel, mesh=_mesh(), compiler_params=_SC_PARAMS,
        out_type=[jax.ShapeDtypeStruct((256, 16), jnp.float32)],
        scratch_types=[pltpu.VMEM((CH_E,), jnp.int32),
                       pltpu.VMEM((8, 16), jnp.float32),
                       pltpu.SemaphoreType.DMA])
    def hist(dst_hbm, hist_hbm, dbuf, hcnt, sem):
        wid = lax.axis_index("s") * 2 + lax.axis_index("c")
        off_u = wid * U_PER_W + jnp.minimum(wid, 16)
        nu = U_PER_W + jnp.where(wid < 16, 1, 0)
        lane = lax.iota(jnp.int32, 16)
        zf = jnp.zeros((16,), jnp.float32)
        for i in range(8):
            hcnt[i, :] = zf

        nch = (nu + 127) // 128

        def chunk(j, _):
            ubase = off_u + j * 128
            pltpu.sync_copy(dst_hbm.at[pl.ds(ubase * 16, CH_E)], dbuf)
            uvalid = jnp.minimum(128, nu - j * 128)

            def unit(u, _):
                d = dbuf[pl.ds(u * 16, 16)]
                rngv = d >> 9
                for e in range(16):
                    rg = rngv[e]
                    cr = rg >> 4
                    oh = (lane == (rg & 15)).astype(jnp.float32)
                    plsc.addupdate(hcnt.at[cr, :], oh)
                return 0
            lax.fori_loop(0, uvalid, unit, 0)
            return 0

        lax.fori_loop(0, nch, chunk, 0)
        pltpu.sync_copy(hcnt.at[pl.ds(0, 8)], hist_hbm.at[pl.ds(wid * 8, 8)])

    return hist


def _make_place():
    @functools.partial(
        pl.kernel, mesh=_mesh(), compiler_params=_SC_PARAMS,
        out_type=[jax.ShapeDtypeStruct((EP2,), jnp.int32),
                  jax.ShapeDtypeStruct((EP2,), jnp.int32)],
        scratch_types=[pltpu.VMEM((CH_E,), jnp.int32),
                       pltpu.VMEM((CH_E,), jnp.int32),
                       pltpu.VMEM((NR * SROW + 16,), jnp.int32),
                       pltpu.VMEM((NR * SROW + 16,), jnp.int32),
                       pltpu.VMEM((1600,), jnp.int32),
                       pltpu.VMEM((1600,), jnp.int32),
                       pltpu.SemaphoreType.DMA,
                       pltpu.SemaphoreType.DMA,
                       pltpu.SemaphoreType.DMA])
    def place(src_hbm, dst_hbm, starts_hbm, psrc_hbm, pdl_hbm,
              sbuf, dbuf, stg_s, stg_d, fill, wcur, sem_i, sem_fs, sem_fd):
        wid = lax.axis_index("s") * 2 + lax.axis_index("c")
        off_u = wid * U_PER_W + jnp.minimum(wid, 16)
        nu = U_PER_W + jnp.where(wid < 16, 1, 0)
        lane = lax.iota(jnp.int32, 16)
        lane0 = lane == 0
        zi = jnp.zeros((16,), jnp.int32)

        def zfill(i, _):
            fill[pl.ds(i * 16, 16)] = zi
            return 0
        lax.fori_loop(0, 100, zfill, 0)
        pltpu.sync_copy(starts_hbm.at[pl.ds(wid * 1600, 1600)], wcur)

        def flush(rg, nun):
            co = pl.multiple_of(wcur[pl.ds(rg * 16, 16)][0], 16)

            def fire(u2, _):
                pltpu.async_copy(
                    stg_s.at[pl.ds(rg * SROW + u2 * 16, 16)],
                    psrc_hbm.at[pl.ds(co + u2 * 16, 16)], sem_fs)
                pltpu.async_copy(
                    stg_d.at[pl.ds(rg * SROW + u2 * 16, 16)],
                    pdl_hbm.at[pl.ds(co + u2 * 16, 16)], sem_fd)
                return 0
            lax.fori_loop(0, nun, fire, 0)

            def drain(u2, _):
                pltpu.make_async_copy(
                    stg_s.at[pl.ds(rg * SROW + u2 * 16, 16)],
                    psrc_hbm.at[pl.ds(co + u2 * 16, 16)], sem_fs).wait()
                pltpu.make_async_copy(
                    stg_d.at[pl.ds(rg * SROW + u2 * 16, 16)],
                    pdl_hbm.at[pl.ds(co + u2 * 16, 16)], sem_fd).wait()
                return 0
            lax.fori_loop(0, nun, drain, 0)
            plsc.store_scatter(wcur, [_splat(rg * 16)],
                               _splat(co + nun * 16), mask=lane0)

        nch = (nu + 127) // 128

        def chunk(j, _):
            ubase = off_u + j * 128
            pltpu.sync_copy(src_hbm.at[pl.ds(ubase * 16, CH_E)], sbuf)
            pltpu.sync_copy(dst_hbm.at[pl.ds(ubase * 16, CH_E)], dbuf)
            uvalid = jnp.minimum(128, nu - j * 128)

            def unit(u, _):
                u16 = u * 16
                d = dbuf[pl.ds(u16, 16)]
                rgv = d >> 9
                srg, perm = plsc.sort_key_val(rgv, lane)
                prev = srg.at[jnp.maximum(lane - 1, 0)].get(
                    mode="promise_in_bounds")
                runstart = (lane == 0) | (srg != prev)
                sidx = jnp.where(runstart, lane, 0)
                runpos = lane - plsc.cummax(sidx)
                base = plsc.load_gather(fill, [srg * 16])
                posv = srg * SROW + base + runpos
                sv = plsc.load_gather(sbuf, [u16 + perm])
                dlv = plsc.load_gather(dbuf, [u16 + perm]) & (RB - 1)
                plsc.store_scatter(stg_s, [posv], sv)
                plsc.store_scatter(stg_d, [posv], dlv)
                nxt = srg.at[jnp.minimum(lane + 1, 15)].get(
                    mode="promise_in_bounds")
                lastrun = jnp.where((lane == 15) | (srg != nxt), 1, 0)
                newf = base + runpos + 1
                plsc.store_scatter(fill, [srg * 16], newf,
                                   mask=lastrun == 1)
                mx = lax.reduce_max(newf, axes=(0,))

                @pl.when(mx >= SD)
                def _():
                    for e in range(16):
                        rg_e = srg[e]

                        @pl.when((lastrun[e] == 1) & (newf[e] >= SD))
                        def _():
                            flush(rg_e, SD // 16)
                            stg_s[pl.ds(rg_e * SROW, 16)] = \
                                stg_s[pl.ds(rg_e * SROW + SD, 16)]
                            stg_d[pl.ds(rg_e * SROW, 16)] = \
                                stg_d[pl.ds(rg_e * SROW + SD, 16)]
                            plsc.store_scatter(fill, [_splat(rg_e * 16)],
                                               _splat(newf[e] - SD),
                                               mask=lane0)
                return 0
            lax.fori_loop(0, uvalid, unit, 0)
            return 0

        lax.fori_loop(0, nch, chunk, 0)

        def tail(rg, _):
            f = fill[pl.ds(rg * 16, 16)][0]
            stg_s[pl.ds(rg * SROW + f, 16)] = zi
            stg_d[pl.ds(rg * SROW + f, 16)] = _splat(DUMMY)

            @pl.when(f > 0)
            def _():
                flush(rg, (f + 15) >> 4)
            return 0
        lax.fori_loop(0, NR, tail, 0)

    return place


def _make_segsum(with_counts):
    @functools.partial(
        pl.kernel, mesh=_mesh(), compiler_params=_SC_PARAMS,
        out_type=[jax.ShapeDtypeStruct((NPAD, H), jnp.float32),
                  jax.ShapeDtypeStruct((NR * 32, 16), jnp.float32)],
        scratch_types=[pltpu.VMEM((S2_CH + 128,), jnp.int32),
                       pltpu.VMEM((S2_CH + 128,), jnp.int32),
                       pltpu.VMEM((2, G, H), jnp.float32),
                       pltpu.VMEM((RB + 1, H), jnp.float32),
                       pltpu.VMEM((40, 16), jnp.float32),
                       pltpu.VMEM((1600,), jnp.int32),
                       pltpu.SemaphoreType.DMA,
                       pltpu.SemaphoreType.DMA,
                       pltpu.SemaphoreType.DMA])
    def seg(x_hbm, psrc_hbm, pdl_hbm, bounds_hbm, sums_hbm, cnts_hbm,
            sbuf, dbuf, rows, acc, cnt16, bbuf, sem_i, sg0, sg1):
        wid = lax.axis_index("s") * 2 + lax.axis_index("c")
        lane = lax.iota(jnp.int32, 16)
        zf = jnp.zeros((16,), jnp.float32)
        zi = jnp.zeros((16,), jnp.int32)
        pltpu.sync_copy(bounds_hbm, bbuf)

        def issue(g, b):
            pltpu.async_copy(
                x_hbm.at[sbuf.at[pl.ds(g * G, G)]], rows.at[b],
                sg0 if b == 0 else sg1)

        def wait(b):
            pltpu.make_async_copy(
                x_hbm.at[pl.ds(0, G)], rows.at[b],
                sg0 if b == 0 else sg1).wait()

        def accum(g, b):
            for sub in range(G // 16):
                dvv = dbuf[pl.ds(g * G + sub * 16, 16)]
                for e in range(16):
                    dloc = dvv[e]
                    rr = sub * 16 + e
                    for f in range(H // 16):
                        sl = pl.ds(f * 16, 16)
                        plsc.addupdate(acc.at[dloc, sl], rows[b, rr, sl])
                    if with_counts:
                        oh = (lane == (dloc & 15)).astype(jnp.float32)
                        plsc.addupdate(cnt16.at[dloc >> 4, :], oh)

        def process_range(r):
            b0 = pl.multiple_of(bbuf[pl.ds(r * 16, 16)][0], 16)
            b1 = pl.multiple_of(bbuf[pl.ds(r * 16 + 16, 16)][0], 16)

            def zero_body(i, _):
                for f in range(H // 16):
                    acc[i, pl.ds(f * 16, 16)] = zf
                return 0
            lax.fori_loop(0, RB + 1, zero_body, 0)

            def zero_cnt(i, _):
                cnt16[i, :] = zf
                return 0
            lax.fori_loop(0, 40, zero_cnt, 0)

            nch = (b1 - b0 + S2_CH - 1) // S2_CH

            def chunk(j, _):
                base = b0 + j * S2_CH
                pltpu.sync_copy(psrc_hbm.at[pl.ds(base, S2_CH)],
                                sbuf.at[pl.ds(0, S2_CH)])
                pltpu.sync_copy(pdl_hbm.at[pl.ds(base, S2_CH)],
                                dbuf.at[pl.ds(0, S2_CH)])
                nvalid = jnp.minimum(S2_CH, b1 - base)
                for q in range(G // 16 - 1):
                    sbuf[pl.ds(nvalid + q * 16, 16)] = zi
                    dbuf[pl.ds(nvalid + q * 16, 16)] = _splat(DUMMY)
                ngrp = (nvalid + G - 1) // G

                issue(0, 0)

                def gpair(gp, _):
                    g0 = 2 * gp

                    @pl.when(g0 + 1 < ngrp)
                    def _():
                        issue(g0 + 1, 1)
                    wait(0)
                    accum(g0, 0)

                    @pl.when(g0 + 1 < ngrp)
                    def _():
                        @pl.when(g0 + 2 < ngrp)
                        def _():
                            issue(g0 + 2, 0)
                        wait(1)
                        accum(g0 + 1, 1)
                    return 0
                lax.fori_loop(0, (ngrp + 1) // 2, gpair, 0)
                return 0
            lax.fori_loop(0, nch, chunk, 0)

            pltpu.sync_copy(acc.at[pl.ds(0, RB)],
                            sums_hbm.at[pl.ds(r * RB, RB)])
            pltpu.sync_copy(cnt16.at[pl.ds(0, 32)],
                            cnts_hbm.at[pl.ds(r * 32, 32)])

        def kloop(k, _):
            r = wid + 32 * k

            @pl.when(r < NR)
            def _():
                process_range(r)
            return 0
        lax.fori_loop(0, 4, kloop, 0)

    return seg


_hist = _make_hist()
_place = _make_place()
_segsum_wc = _make_segsum(True)
_segsum_nc = _make_segsum(False)


def _partition(src, dst):
    src_p = jnp.pad(src, (0, EPAD_IN - E))
    dst_p = jnp.pad(dst, (0, EPAD_IN - E))
    hist = _hist(dst_p)[0]
    cnt_wr = hist.reshape(32, 8, 16)[:, :7, :].reshape(32, 112)[:, :NR]
    cnt_wr = cnt_wr.astype(jnp.int32)
    pc = ((cnt_wr + 15) // 16) * 16
    p_r = pc.sum(axis=0)
    range_starts = jnp.concatenate(
        [jnp.zeros((1,), jnp.int32), jnp.cumsum(p_r)]).astype(jnp.int32)
    starts_wr = range_starts[None, :NR] + (jnp.cumsum(pc, axis=0) - pc)
    starts = jnp.zeros((32, 100, 16), jnp.int32).at[:, :NR, 0].set(starts_wr)
    bounds = jnp.full((100, 16), range_starts[NR], jnp.int32)
    bounds = bounds.at[:NR + 1, 0].set(range_starts)
    psrc, pdl = _place(src_p, dst_p, starts.reshape(-1))
    bounds = bounds.reshape(-1)
    return psrc, pdl, bounds


def _seg(x, part):
    psrc, pdl, bounds = part
    sums, cnts = _segsum_wc(x, psrc, pdl, bounds)
    counts = cnts.reshape(NPAD)[:N_NODE].reshape(N_NODE, 1)
    return sums[:N_NODE], counts


def _seg_nc(x, part):
    psrc, pdl, bounds = part
    return _segsum_nc(x, psrc, pdl, bounds)[0][:N_NODE]


BN = 2000
NB = N_NODE // BN


def _enc_body(x_ref, w_ref, b_ref, o_ref):
    o_ref[...] = jnp.maximum(
        jnp.dot(x_ref[...], w_ref[...], preferred_element_type=jnp.float32)
        + b_ref[...], 0.0)


def _encode(x, w, b):
    return pl.pallas_call(
        _enc_body,
        grid=(NB,),
        in_specs=[
            pl.BlockSpec((BN, D_IN), lambda i: (i, 0)),
            pl.BlockSpec((D_IN, H), lambda i: (0, 0)),
            pl.BlockSpec((1, H), lambda i: (0, 0)),
        ],
        out_specs=pl.BlockSpec((BN, H), lambda i: (i, 0)),
        out_shape=jax.ShapeDtypeStruct((N_NODE, H), jnp.float32),
    )(x, w, b.reshape(1, H))


def _inv_counts(c_ref):
    return 1.0 / jnp.maximum(c_ref[...], 1.0)


def _l1bus_body(sbb_ref, ibb_ref, sdb_ref, idb_ref, xb_ref,
                wlbb_ref, wrbb_ref, wldb_ref, wrdb_ref, bias_ref, o_ref):
    mbb = (sbb_ref[...] * _inv_counts(ibb_ref)) @ wlbb_ref[...]
    mdb = (sdb_ref[...] * _inv_counts(idb_ref)) @ wldb_ref[...]
    xr = xb_ref[...] @ (wrbb_ref[...] + wrdb_ref[...])
    o_ref[...] = jnp.maximum(mbb + mdb + xr + bias_ref[...], 0.0)


def _l1dev_body(s_ref, i_ref, x_ref, wl_ref, wr_ref, bias_ref, o_ref):
    m = (s_ref[...] * _inv_counts(i_ref)) @ wl_ref[...]
    o_ref[...] = jnp.maximum(m + x_ref[...] @ wr_ref[...] + bias_ref[...], 0.0)


def _l2head_body(sbb_ref, ibb_ref, sdb_ref, idb_ref, xb_ref,
                 wlbb_ref, wrbb_ref, wldb_ref, wrdb_ref, bias_ref,
                 hw_ref, hb_ref, o_ref, acc_ref):
    i = pl.program_id(0)

    @pl.when(i == 0)
    def _init():
        acc_ref[...] = jnp.zeros_like(acc_ref)

    mbb = (sbb_ref[...] * _inv_counts(ibb_ref)) @ wlbb_ref[...]
    mdb = (sdb_ref[...] * _inv_counts(idb_ref)) @ wldb_ref[...]
    xr = xb_ref[...] @ (wrbb_ref[...] + wrdb_ref[...])
    blk = jnp.maximum(mbb + mdb + xr + bias_ref[...], 0.0)
    acc_ref[...] += jnp.sum(blk, axis=0, keepdims=True)

    @pl.when(i == NB - 1)
    def _fin():
        o_ref[...] = (acc_ref[...] / N_NODE) @ hw_ref[...] + hb_ref[...]


_SPEC_NH = pl.BlockSpec((BN, H), lambda i: (i, 0))
_SPEC_C = pl.BlockSpec((BN, 1), lambda i: (i, 0))
_SPEC_W = pl.BlockSpec((H, H), lambda i: (0, 0))
_SPEC_B = pl.BlockSpec((1, H), lambda i: (0, 0))


def _layer_bus(sbb, ibb, sdb, idb, xb, wlbb, wrbb, wldb, wrdb, bias):
    return pl.pallas_call(
        _l1bus_body,
        grid=(NB,),
        in_specs=[_SPEC_NH, _SPEC_C, _SPEC_NH, _SPEC_C, _SPEC_NH,
                  _SPEC_W, _SPEC_W, _SPEC_W, _SPEC_W, _SPEC_B],
        out_specs=_SPEC_NH,
        out_shape=jax.ShapeDtypeStruct((N_NODE, H), jnp.float32),
    )(sbb, ibb, sdb, idb, xb, wlbb, wrbb, wldb, wrdb, bias.reshape(1, H))


def _layer_dev(s, iv, x, wl, wr, bias):
    return pl.pallas_call(
        _l1dev_body,
        grid=(NB,),
        in_specs=[_SPEC_NH, _SPEC_C, _SPEC_NH, _SPEC_W, _SPEC_W, _SPEC_B],
        out_specs=_SPEC_NH,
        out_shape=jax.ShapeDtypeStruct((N_NODE, H), jnp.float32),
    )(s, iv, x, wl, wr, bias.reshape(1, H))


def _layer2_head(sbb, ibb, sdb, idb, xb2, wlbb, wrbb, wldb, wrdb, bias,
                 hw, hb):
    return pl.pallas_call(
        _l2head_body,
        grid=(NB,),
        in_specs=[_SPEC_NH, _SPEC_C, _SPEC_NH, _SPEC_C, _SPEC_NH,
                  _SPEC_W, _SPEC_W, _SPEC_W, _SPEC_W, _SPEC_B,
                  pl.BlockSpec((H, 1), lambda i: (0, 0)),
                  pl.BlockSpec((1, 1), lambda i: (0, 0))],
        out_specs=pl.BlockSpec((1, 1), lambda i: (0, 0)),
        out_shape=jax.ShapeDtypeStruct((1, 1), jnp.float32),
        scratch_shapes=[pltpu.VMEM((1, H), jnp.float32)],
    )(sbb, ibb, sdb, idb, xb2, wlbb, wrbb, wldb, wrdb, bias.reshape(1, H),
      hw, hb.reshape(1, 1))


def kernel(x_bus, x_device, edge_index_bb, edge_index_bd, edge_index_db,
           bus_W, bus_b, dev_W, dev_b,
           Wl_bb1, bl_bb1, Wr_bb1, Wl_bb2, bl_bb2, Wr_bb2,
           Wl_bd1, bl_bd1, Wr_bd1, Wl_bd2, bl_bd2, Wr_bd2,
           Wl_db1, bl_db1, Wr_db1, Wl_db2, bl_db2, Wr_db2,
           head_W, head_b):
    xb = _encode(x_bus, bus_W, bus_b)
    xd = _encode(x_device, dev_W, dev_b)

    part_bb = _partition(edge_index_bb[0], edge_index_bb[1])
    part_bd = _partition(edge_index_bd[0], edge_index_bd[1])
    part_db = _partition(edge_index_db[0], edge_index_db[1])

    s_bb1, c_bb = _seg(xb, part_bb)
    s_bd1, c_bd = _seg(xb, part_bd)
    s_db1, c_db = _seg(xd, part_db)

    xb2 = _layer_bus(s_bb1, c_bb, s_db1, c_db, xb,
                     Wl_bb1, Wr_bb1, Wl_db1, Wr_db1, bl_bb1 + bl_db1)
    xd2 = _layer_dev(s_bd1, c_bd, xd, Wl_bd1, Wr_bd1, bl_bd1)

    s_bb2 = _seg_nc(xb2, part_bb)
    s_db2 = _seg_nc(xd2, part_db)

    return _layer2_head(s_bb2, c_bb, s_db2, c_db, xb2,
                        Wl_bb2, Wr_bb2, Wl_db2, Wr_db2, bl_bb2 + bl_db2,
                        head_W, head_b)
